# baseline (speedup 1.0000x reference)
import functools

import jax
import jax.numpy as jnp
from jax import lax
from jax.experimental import pallas as pl
from jax.experimental.pallas import tpu as pltpu
from jax.experimental.pallas import tpu_sc as plsc

D_MODEL = 768
NUM_EXPERTS = 64
EXPERT_DIM = 256
CAP = 128
N_TOKENS = 2048
TBLK = 256
NB = N_TOKENS // TBLK
RCHUNK = 32
S_SLOTS = 4096
NCHUNK = S_SLOTS // RCHUNK

NC = 2
NS = 16
L = 16
NW = NC * NS
SLOTS_PER_W = S_SLOTS // NW
TOK_PER_W = N_TOKENS // NW


def _router_body(x_ref, gw_ref, bias_ref, pcode_ref, wgt_ref, cnt_out_ref,
                 cnt_ref):
    b = pl.program_id(0)

    @pl.when(b == 0)
    def _():
        cnt_ref[...] = jnp.zeros_like(cnt_ref)

    xb = x_ref[...]
    gw = gw_ref[...]
    logits = lax.dot_general(xb, gw, (((1,), (1,)), ((), ())),
                             preferred_element_type=jnp.float32)
    logits = logits + bias_ref[...]
    m = jnp.max(logits, axis=1, keepdims=True)
    s = jnp.sum(jnp.exp(logits - m), axis=1)
    p = 1.0 / s
    w = p / (p + 1e-8)

    col = lax.broadcasted_iota(jnp.int32, (TBLK, NUM_EXPERTS), 1)
    sel = jnp.min(jnp.where(logits == m, col, NUM_EXPERTS), axis=1)
    onehot = (col == sel[:, None]).astype(jnp.float32)

    c = onehot
    d = 1
    while d < TBLK:
        shifted = jnp.concatenate(
            [jnp.zeros((d, NUM_EXPERTS), jnp.float32), c[: TBLK - d]], axis=0)
        c = c + shifted
        d *= 2

    run = cnt_ref[...]
    rank = jnp.sum(onehot * (c + run), axis=1) - 1.0
    new_run = run + jnp.sum(onehot, axis=0, keepdims=True)
    cnt_ref[...] = new_run
    cnt_out_ref[...] = new_run.astype(jnp.int32)

    rank_i = rank.astype(jnp.int32)
    pcode = sel * N_TOKENS + rank_i
    wgt = jnp.where(rank_i >= CAP, 0.0, w)
    pcode_ref[...] = pcode.reshape(1, 1, TBLK)
    wgt_ref[...] = wgt.reshape(1, 1, TBLK)


def _router(xf, gate_w, expert_bias):
    pcode3, wgt3, cnt = pl.pallas_call(
        _router_body,
        grid=(NB,),
        in_specs=[
            pl.BlockSpec((TBLK, D_MODEL), lambda b: (b, 0)),
            pl.BlockSpec((NUM_EXPERTS, D_MODEL), lambda b: (0, 0)),
            pl.BlockSpec((1, NUM_EXPERTS), lambda b: (0, 0)),
        ],
        out_specs=[
            pl.BlockSpec((1, 1, TBLK), lambda b: (b, 0, 0)),
            pl.BlockSpec((1, 1, TBLK), lambda b: (b, 0, 0)),
            pl.BlockSpec((1, NUM_EXPERTS), lambda b: (0, 0)),
        ],
        out_shape=[
            jax.ShapeDtypeStruct((NB, 1, TBLK), jnp.int32),
            jax.ShapeDtypeStruct((NB, 1, TBLK), jnp.float32),
            jax.ShapeDtypeStruct((1, NUM_EXPERTS), jnp.int32),
        ],
        scratch_shapes=[pltpu.VMEM((1, NUM_EXPERTS), jnp.float32)],
    )(xf, gate_w, expert_bias.reshape(1, NUM_EXPERTS))
    return (pcode3.reshape(N_TOKENS), wgt3.reshape(N_TOKENS),
            cnt.reshape(NUM_EXPERTS))


def _dispatch_body(pcode_hbm, cnt_hbm, x_hbm,
                   xs_hbm, slot_hbm, offs_hbm, nch_hbm,
                   pos_v, cnt_v, offs_v, tok_v, slot_v, nch_v, rows_v, sem):
    wid = lax.axis_index("s") * NC + lax.axis_index("c")
    base = wid * SLOTS_PER_W
    pltpu.sync_copy(pcode_hbm, pos_v)
    pltpu.sync_copy(cnt_hbm, cnt_v)

    lanes = lax.broadcasted_iota(jnp.int32, (L,), 0)

    carry = jnp.int32(0)
    ends = []
    for k in range(NUM_EXPERTS // L):
        cnt_k = cnt_v[pl.ds(k * L, L)]
        padded = ((jnp.minimum(cnt_k, CAP) + (RCHUNK - 1)) >> 5) << 5
        excl = plsc.cumsum(padded) - padded + carry
        offs_v[pl.ds(k * L, L)] = excl
        ends.append((excl + padded) >> 5)
        carry = carry + jnp.sum(padded)

    for k in range(NUM_EXPERTS // L):
        cnt_k = cnt_v[pl.ds(k * L, L)]
        padded = ((jnp.minimum(cnt_k, CAP) + (RCHUNK - 1)) >> 5) << 5
        nch_v[pl.ds(k * L, L)] = padded >> 5

    for j in range(SLOTS_PER_W // L):
        tok_v[pl.ds(j * L, L)] = (base + j * L + lanes) & (N_TOKENS - 1)

    def scatter_step(i, carry):
        pc = pos_v[pl.ds(i * L, L)]
        sel = pc >> 11
        rank = pc & (N_TOKENS - 1)
        slot = plsc.load_gather(offs_v, [sel]) + rank
        valid = rank < CAP
        slot_v[pl.ds(i * L, L)] = jnp.where(valid, slot, 0)
        m = valid & (slot >= base) & (slot < base + SLOTS_PER_W)
        plsc.store_scatter(tok_v, [jnp.where(m, slot - base, 0)],
                           lanes + i * L, mask=m)
        return carry

    lax.fori_loop(0, N_TOKENS // L, scatter_step, 0)

    @pl.when(wid == 0)
    def _():
        pltpu.sync_copy(slot_v, slot_hbm)
        pltpu.sync_copy(offs_v, offs_hbm)
        pltpu.sync_copy(nch_v, nch_hbm)

    pltpu.async_copy(x_hbm.at[tok_v], rows_v, sem).wait()
    pltpu.sync_copy(rows_v, xs_hbm.at[pl.ds(base, SLOTS_PER_W)])


def _dispatch(pcode, counts, xf):
    mesh = plsc.VectorSubcoreMesh(core_axis_name="c", subcore_axis_name="s")
    f = functools.partial(
        pl.kernel,
        mesh=mesh,
        out_type=(
            jax.ShapeDtypeStruct((S_SLOTS, D_MODEL), jnp.float32),
            jax.ShapeDtypeStruct((N_TOKENS,), jnp.int32),
            jax.ShapeDtypeStruct((NUM_EXPERTS,), jnp.int32),
            jax.ShapeDtypeStruct((NUM_EXPERTS,), jnp.int32),
        ),
        compiler_params=pltpu.CompilerParams(needs_layout_passes=False),
        scratch_types=[
            pltpu.VMEM((N_TOKENS,), jnp.int32),
            pltpu.VMEM((NUM_EXPERTS,), jnp.int32),
            pltpu.VMEM((NUM_EXPERTS,), jnp.int32),
            pltpu.VMEM((SLOTS_PER_W,), jnp.int32),
            pltpu.VMEM((N_TOKENS,), jnp.int32),
            pltpu.VMEM((NUM_EXPERTS,), jnp.int32),
            pltpu.VMEM((SLOTS_PER_W, D_MODEL), jnp.float32),
            pltpu.SemaphoreType.DMA,
        ],
    )(_dispatch_body)
    return f(pcode, counts, xf)


def _experts_body(offs_ref, nch_ref, xs_ref, wg_ref, wv_ref, wo_ref, h_ref):
    e = pl.program_id(0)
    off = offs_ref[e]
    n = nch_ref[e]
    wg = wg_ref[0]
    wv = wv_ref[0]
    wo = wo_ref[0]
    dn = (((1,), (1,)), ((), ()))
    for j in range(CAP // RCHUNK):
        @pl.when(j < n)
        def _():
            row = pl.multiple_of(off + j * RCHUNK, RCHUNK)
            xe = xs_ref[pl.ds(row, RCHUNK), :]
            g = lax.dot_general(xe, wg, dn,
                                preferred_element_type=jnp.float32)
            v = lax.dot_general(xe, wv, dn,
                                preferred_element_type=jnp.float32)
            u = (g / (1.0 + jnp.exp(-g))) * v
            h_ref[pl.ds(row, RCHUNK), :] = lax.dot_general(
                u, wo, dn, preferred_element_type=jnp.float32)


def _experts(offs, nch, xs, w_gate, w_value, w_out):
    grid_spec = pltpu.PrefetchScalarGridSpec(
        num_scalar_prefetch=2,
        grid=(NUM_EXPERTS,),
        in_specs=[
            pl.BlockSpec((S_SLOTS, D_MODEL), lambda e, offs, nch: (0, 0)),
            pl.BlockSpec((1, EXPERT_DIM, D_MODEL),
                         lambda e, offs, nch: (e, 0, 0)),
            pl.BlockSpec((1, EXPERT_DIM, D_MODEL),
                         lambda e, offs, nch: (e, 0, 0)),
            pl.BlockSpec((1, D_MODEL, EXPERT_DIM),
                         lambda e, offs, nch: (e, 0, 0)),
        ],
        out_specs=pl.BlockSpec((S_SLOTS, D_MODEL), lambda e, offs, nch: (0, 0)),
    )
    return pl.pallas_call(
        _experts_body,
        grid_spec=grid_spec,
        out_shape=jax.ShapeDtypeStruct((S_SLOTS, D_MODEL), jnp.float32),
        compiler_params=pltpu.CompilerParams(
            dimension_semantics=("arbitrary",)),
    )(offs, nch, xs, w_gate, w_value, w_out)


def _combine_body(slot_hbm, wgt_hbm, h_hbm, out_hbm, pos_v, wgt_v, rows_v, sem):
    wid = lax.axis_index("s") * NC + lax.axis_index("c")
    tb = wid * TOK_PER_W
    pltpu.sync_copy(slot_hbm.at[pl.ds(tb, TOK_PER_W)], pos_v)
    pltpu.sync_copy(wgt_hbm.at[pl.ds(tb, TOK_PER_W)], wgt_v)
    pltpu.async_copy(h_hbm.at[pos_v], rows_v, sem).wait()

    def scale_row(i, carry):
        wv = plsc.load_gather(wgt_v, [jnp.broadcast_to(i, (L,))])
        for j in range(D_MODEL // L):
            sl = pl.ds(j * L, L)
            rows_v[i, sl] = rows_v[i, sl] * wv
        return carry

    lax.fori_loop(0, TOK_PER_W, scale_row, 0)
    pltpu.sync_copy(rows_v, out_hbm.at[pl.ds(tb, TOK_PER_W)])


def _combine(slot, wgt, h):
    mesh = plsc.VectorSubcoreMesh(core_axis_name="c", subcore_axis_name="s")
    f = functools.partial(
        pl.kernel,
        mesh=mesh,
        out_type=jax.ShapeDtypeStruct((N_TOKENS, D_MODEL), jnp.float32),
        compiler_params=pltpu.CompilerParams(needs_layout_passes=False),
        scratch_types=[
            pltpu.VMEM((TOK_PER_W,), jnp.int32),
            pltpu.VMEM((TOK_PER_W,), jnp.float32),
            pltpu.VMEM((TOK_PER_W, D_MODEL), jnp.float32),
            pltpu.SemaphoreType.DMA,
        ],
    )(_combine_body)
    return f(slot, wgt, h)


def kernel(x, gate_w, expert_bias, w_gate, w_value, w_out):
    B_, T_, D_ = x.shape
    xf = x.reshape(T_ * B_, D_)
    pcode, wgt, counts = _router(xf, gate_w, expert_bias)
    xs, slot, offs, nch = _dispatch(pcode, counts, xf)
    h = _experts(offs, nch, xs, w_gate, w_value, w_out)
    out = _combine(slot, wgt, h)
    return out.reshape(B_, T_, D_)


# speedup vs baseline: 1.0184x; 1.0184x over previous
import functools

import jax
import jax.numpy as jnp
from jax import lax
from jax.experimental import pallas as pl
from jax.experimental.pallas import tpu as pltpu
from jax.experimental.pallas import tpu_sc as plsc

D_MODEL = 768
NUM_EXPERTS = 64
EXPERT_DIM = 256
CAP = 128
N_TOKENS = 2048
TBLK = 256
NB = N_TOKENS // TBLK
S_SLOTS = NUM_EXPERTS * CAP

NC = 2
NS = 16
L = 16
NW = NC * NS
SLOTS_PER_W = S_SLOTS // NW
GCHUNK = 128
TOK_PER_W = N_TOKENS // NW


def _router_body(x_ref, gw_ref, bias_ref, pcode_ref, wgt_ref, cnt_ref):
    b = pl.program_id(0)

    @pl.when(b == 0)
    def _():
        cnt_ref[...] = jnp.zeros_like(cnt_ref)

    xb = x_ref[...]
    gw = gw_ref[...]
    logits = lax.dot_general(xb, gw, (((1,), (1,)), ((), ())),
                             preferred_element_type=jnp.float32)
    logits = logits + bias_ref[...]
    m = jnp.max(logits, axis=1, keepdims=True)
    s = jnp.sum(jnp.exp(logits - m), axis=1)
    p = 1.0 / s
    w = p / (p + 1e-8)

    col = lax.broadcasted_iota(jnp.int32, (TBLK, NUM_EXPERTS), 1)
    sel = jnp.min(jnp.where(logits == m, col, NUM_EXPERTS), axis=1)
    onehot = (col == sel[:, None]).astype(jnp.float32)

    c = onehot
    d = 1
    while d < TBLK:
        shifted = jnp.concatenate(
            [jnp.zeros((d, NUM_EXPERTS), jnp.float32), c[: TBLK - d]], axis=0)
        c = c + shifted
        d *= 2

    run = cnt_ref[...]
    rank = jnp.sum(onehot * (c + run), axis=1) - 1.0
    cnt_ref[...] = run + jnp.sum(onehot, axis=0, keepdims=True)

    rank_i = rank.astype(jnp.int32)
    pcode = sel * N_TOKENS + rank_i
    wgt = jnp.where(rank_i >= CAP, 0.0, w)
    pcode_ref[...] = pcode.reshape(1, 1, TBLK)
    wgt_ref[...] = wgt.reshape(1, 1, TBLK)


def _router(xf, gate_w, expert_bias):
    pcode3, wgt3 = pl.pallas_call(
        _router_body,
        grid=(NB,),
        in_specs=[
            pl.BlockSpec((TBLK, D_MODEL), lambda b: (b, 0)),
            pl.BlockSpec((NUM_EXPERTS, D_MODEL), lambda b: (0, 0)),
            pl.BlockSpec((1, NUM_EXPERTS), lambda b: (0, 0)),
        ],
        out_specs=[
            pl.BlockSpec((1, 1, TBLK), lambda b: (b, 0, 0)),
            pl.BlockSpec((1, 1, TBLK), lambda b: (b, 0, 0)),
        ],
        out_shape=[
            jax.ShapeDtypeStruct((NB, 1, TBLK), jnp.int32),
            jax.ShapeDtypeStruct((NB, 1, TBLK), jnp.float32),
        ],
        scratch_shapes=[pltpu.VMEM((1, NUM_EXPERTS), jnp.float32)],
    )(xf, gate_w, expert_bias.reshape(1, NUM_EXPERTS))
    return pcode3.reshape(N_TOKENS), wgt3.reshape(N_TOKENS)


def _dispatch_body(pcode_hbm, x_hbm, xs_hbm, slot_hbm,
                   pos_v, tok0_v, tok1_v, slot_v, rows_v, sem):
    wid = lax.axis_index("s") * NC + lax.axis_index("c")
    base = wid * SLOTS_PER_W
    pltpu.sync_copy(pcode_hbm, pos_v)

    lanes = lax.broadcasted_iota(jnp.int32, (L,), 0)

    for j in range(GCHUNK // L):
        fill = (base + j * L + lanes) & (N_TOKENS - 1)
        tok0_v[pl.ds(j * L, L)] = fill
        tok1_v[pl.ds(j * L, L)] = (fill + GCHUNK) & (N_TOKENS - 1)

    def scatter_step(i, carry):
        pc = pos_v[pl.ds(i * L, L)]
        sel = pc >> 11
        rank = pc & (N_TOKENS - 1)
        valid = rank < CAP
        slot = sel * CAP + rank
        slot_v[pl.ds(i * L, L)] = jnp.where(valid, slot, 0)
        rel = slot - base
        m0 = valid & (rel >= 0) & (rel < GCHUNK)
        m1 = valid & (rel >= GCHUNK) & (rel < SLOTS_PER_W)
        toks = lanes + i * L
        plsc.store_scatter(tok0_v, [jnp.where(m0, rel, 0)], toks, mask=m0)
        plsc.store_scatter(tok1_v, [jnp.where(m1, rel - GCHUNK, 0)], toks,
                           mask=m1)
        return carry

    lax.fori_loop(0, N_TOKENS // L, scatter_step, 0)

    @pl.when(wid == 0)
    def _():
        pltpu.sync_copy(slot_v, slot_hbm)

    for c, tok_v in enumerate((tok0_v, tok1_v)):
        pltpu.async_copy(x_hbm.at[tok_v], rows_v, sem).wait()
        pltpu.sync_copy(rows_v, xs_hbm.at[pl.ds(base + c * GCHUNK, GCHUNK)])


def _dispatch(pcode, xf):
    mesh = plsc.VectorSubcoreMesh(core_axis_name="c", subcore_axis_name="s")
    f = functools.partial(
        pl.kernel,
        mesh=mesh,
        out_type=(
            jax.ShapeDtypeStruct((S_SLOTS, D_MODEL), jnp.float32),
            jax.ShapeDtypeStruct((N_TOKENS,), jnp.int32),
        ),
        compiler_params=pltpu.CompilerParams(needs_layout_passes=False),
        scratch_types=[
            pltpu.VMEM((N_TOKENS,), jnp.int32),
            pltpu.VMEM((GCHUNK,), jnp.int32),
            pltpu.VMEM((GCHUNK,), jnp.int32),
            pltpu.VMEM((N_TOKENS,), jnp.int32),
            pltpu.VMEM((GCHUNK, D_MODEL), jnp.float32),
            pltpu.SemaphoreType.DMA,
        ],
    )(_dispatch_body)
    return f(pcode, xf)


def _experts_body(xs_ref, wg_ref, wv_ref, wo_ref, h_ref):
    xe = xs_ref[...]
    wg = wg_ref[0]
    wv = wv_ref[0]
    wo = wo_ref[0]
    g = lax.dot_general(xe, wg, (((1,), (1,)), ((), ())),
                        preferred_element_type=jnp.float32)
    v = lax.dot_general(xe, wv, (((1,), (1,)), ((), ())),
                        preferred_element_type=jnp.float32)
    u = (g / (1.0 + jnp.exp(-g))) * v
    h_ref[...] = lax.dot_general(u, wo, (((1,), (1,)), ((), ())),
                                 preferred_element_type=jnp.float32)


def _experts(xs, w_gate, w_value, w_out):
    return pl.pallas_call(
        _experts_body,
        grid=(NUM_EXPERTS,),
        in_specs=[
            pl.BlockSpec((CAP, D_MODEL), lambda e: (e, 0)),
            pl.BlockSpec((1, EXPERT_DIM, D_MODEL), lambda e: (e, 0, 0)),
            pl.BlockSpec((1, EXPERT_DIM, D_MODEL), lambda e: (e, 0, 0)),
            pl.BlockSpec((1, D_MODEL, EXPERT_DIM), lambda e: (e, 0, 0)),
        ],
        out_specs=pl.BlockSpec((CAP, D_MODEL), lambda e: (e, 0)),
        out_shape=jax.ShapeDtypeStruct((S_SLOTS, D_MODEL), jnp.float32),
        compiler_params=pltpu.CompilerParams(
            dimension_semantics=("arbitrary",)),
    )(xs, w_gate, w_value, w_out)


def _combine_body(slot_hbm, wgt_hbm, h_hbm, out_hbm, pos_v, wgt_v, rows_v, sem):
    wid = lax.axis_index("s") * NC + lax.axis_index("c")
    tb = wid * TOK_PER_W
    pltpu.sync_copy(slot_hbm.at[pl.ds(tb, TOK_PER_W)], pos_v)
    pltpu.sync_copy(wgt_hbm.at[pl.ds(tb, TOK_PER_W)], wgt_v)
    pltpu.async_copy(h_hbm.at[pos_v], rows_v, sem).wait()

    def scale_row(i, carry):
        wv = plsc.load_gather(wgt_v, [jnp.broadcast_to(i, (L,))])
        for j in range(D_MODEL // L):
            sl = pl.ds(j * L, L)
            rows_v[i, sl] = rows_v[i, sl] * wv
        return carry

    lax.fori_loop(0, TOK_PER_W, scale_row, 0)
    pltpu.sync_copy(rows_v, out_hbm.at[pl.ds(tb, TOK_PER_W)])


def _combine(slot, wgt, h):
    mesh = plsc.VectorSubcoreMesh(core_axis_name="c", subcore_axis_name="s")
    f = functools.partial(
        pl.kernel,
        mesh=mesh,
        out_type=jax.ShapeDtypeStruct((N_TOKENS, D_MODEL), jnp.float32),
        compiler_params=pltpu.CompilerParams(needs_layout_passes=False),
        scratch_types=[
            pltpu.VMEM((TOK_PER_W,), jnp.int32),
            pltpu.VMEM((TOK_PER_W,), jnp.float32),
            pltpu.VMEM((TOK_PER_W, D_MODEL), jnp.float32),
            pltpu.SemaphoreType.DMA,
        ],
    )(_combine_body)
    return f(slot, wgt, h)


def kernel(x, gate_w, expert_bias, w_gate, w_value, w_out):
    B_, T_, D_ = x.shape
    xf = x.reshape(T_ * B_, D_)
    pcode, wgt = _router(xf, gate_w, expert_bias)
    xs, slot = _dispatch(pcode, xf)
    h = _experts(xs, w_gate, w_value, w_out)
    out = _combine(slot, wgt, h)
    return out.reshape(B_, T_, D_)


# speedup vs baseline: 1.0689x; 1.0496x over previous
import functools

import jax
import jax.numpy as jnp
from jax import lax
from jax.experimental import pallas as pl
from jax.experimental.pallas import tpu as pltpu
from jax.experimental.pallas import tpu_sc as plsc

D_MODEL = 768
NUM_EXPERTS = 64
EXPERT_DIM = 256
CAP = 128
N_TOKENS = 2048
TBLK = 256
NB = N_TOKENS // TBLK
S_SLOTS = NUM_EXPERTS * CAP

NC = 2
NS = 16
L = 16
NW = NC * NS
SLOTS_PER_W = S_SLOTS // NW
GCHUNK = 128
GSUB = 32
TOK_PER_W = N_TOKENS // NW


def _router_body(x_ref, gw_ref, bias_ref, pcode_ref, wgt_ref, cnt_out_ref,
                 cnt_ref):
    b = pl.program_id(0)

    @pl.when(b == 0)
    def _():
        cnt_ref[...] = jnp.zeros_like(cnt_ref)

    xb = x_ref[...]
    gw = gw_ref[...]
    logits = lax.dot_general(xb, gw, (((1,), (1,)), ((), ())),
                             preferred_element_type=jnp.float32)
    logits = logits + bias_ref[...]
    m = jnp.max(logits, axis=1, keepdims=True)
    s = jnp.sum(jnp.exp(logits - m), axis=1)
    p = 1.0 / s
    w = p / (p + 1e-8)

    col = lax.broadcasted_iota(jnp.int32, (TBLK, NUM_EXPERTS), 1)
    sel = jnp.min(jnp.where(logits == m, col, NUM_EXPERTS), axis=1)
    onehot = (col == sel[:, None]).astype(jnp.float32)

    c = onehot
    d = 1
    while d < TBLK:
        shifted = jnp.concatenate(
            [jnp.zeros((d, NUM_EXPERTS), jnp.float32), c[: TBLK - d]], axis=0)
        c = c + shifted
        d *= 2

    run = cnt_ref[...]
    rank = jnp.sum(onehot * (c + run), axis=1) - 1.0
    new_run = run + jnp.sum(onehot, axis=0, keepdims=True)
    cnt_ref[...] = new_run
    cnt_out_ref[...] = new_run.astype(jnp.int32)

    rank_i = rank.astype(jnp.int32)
    pcode = sel * N_TOKENS + rank_i
    wgt = jnp.where(rank_i >= CAP, 0.0, w)
    pcode_ref[...] = pcode.reshape(1, 1, TBLK)
    wgt_ref[...] = wgt.reshape(1, 1, TBLK)


def _router(xf, gate_w, expert_bias):
    pcode3, wgt3, cnt = pl.pallas_call(
        _router_body,
        grid=(NB,),
        in_specs=[
            pl.BlockSpec((TBLK, D_MODEL), lambda b: (b, 0)),
            pl.BlockSpec((NUM_EXPERTS, D_MODEL), lambda b: (0, 0)),
            pl.BlockSpec((1, NUM_EXPERTS), lambda b: (0, 0)),
        ],
        out_specs=[
            pl.BlockSpec((1, 1, TBLK), lambda b: (b, 0, 0)),
            pl.BlockSpec((1, 1, TBLK), lambda b: (b, 0, 0)),
            pl.BlockSpec((1, NUM_EXPERTS), lambda b: (0, 0)),
        ],
        out_shape=[
            jax.ShapeDtypeStruct((NB, 1, TBLK), jnp.int32),
            jax.ShapeDtypeStruct((NB, 1, TBLK), jnp.float32),
            jax.ShapeDtypeStruct((1, NUM_EXPERTS), jnp.int32),
        ],
        scratch_shapes=[pltpu.VMEM((1, NUM_EXPERTS), jnp.float32)],
    )(xf, gate_w, expert_bias.reshape(1, NUM_EXPERTS))
    return (pcode3.reshape(N_TOKENS), wgt3.reshape(N_TOKENS),
            cnt.reshape(NUM_EXPERTS))


def _dispatch_body(pcode_hbm, cnt_hbm, x_hbm, xs_hbm, slot_hbm,
                   pos_v, cnt_v, tok0_v, tok1_v, slot_v, rows_v, sem):
    wid = lax.axis_index("s") * NC + lax.axis_index("c")
    base = wid * SLOTS_PER_W
    pltpu.sync_copy(pcode_hbm, pos_v)
    pltpu.sync_copy(cnt_hbm, cnt_v)

    lanes = lax.broadcasted_iota(jnp.int32, (L,), 0)

    for j in range(GCHUNK // L):
        fill = (base + j * L + lanes) & (N_TOKENS - 1)
        tok0_v[pl.ds(j * L, L)] = fill
        tok1_v[pl.ds(j * L, L)] = (fill + GCHUNK) & (N_TOKENS - 1)

    def scatter_step(i, carry):
        pc = pos_v[pl.ds(i * L, L)]
        sel = pc >> 11
        rank = pc & (N_TOKENS - 1)
        valid = rank < CAP
        slot = sel * CAP + rank
        slot_v[pl.ds(i * L, L)] = jnp.where(valid, slot, 0)
        rel = slot - base
        m0 = valid & (rel >= 0) & (rel < GCHUNK)
        m1 = valid & (rel >= GCHUNK) & (rel < SLOTS_PER_W)
        toks = lanes + i * L
        plsc.store_scatter(tok0_v, [jnp.where(m0, rel, 0)], toks, mask=m0)
        plsc.store_scatter(tok1_v, [jnp.where(m1, rel - GCHUNK, 0)], toks,
                           mask=m1)
        return carry

    lax.fori_loop(0, N_TOKENS // L, scatter_step, 0)

    @pl.when(wid == 0)
    def _():
        pltpu.sync_copy(slot_v, slot_hbm)

    for c, tok_v in enumerate((tok0_v, tok1_v)):
        cnt_e = jnp.max(plsc.load_gather(
            cnt_v, [jnp.broadcast_to(2 * wid + c, (L,))]))
        cnt_e = jnp.minimum(cnt_e, CAP)
        for q in range(GCHUNK // GSUB):
            @pl.when(q * GSUB < cnt_e)
            def _():
                pltpu.async_copy(
                    x_hbm.at[tok_v.at[pl.ds(q * GSUB, GSUB)]],
                    rows_v, sem).wait()
                pltpu.sync_copy(
                    rows_v,
                    xs_hbm.at[pl.ds(base + c * GCHUNK + q * GSUB, GSUB)])


def _dispatch(pcode, counts, xf):
    mesh = plsc.VectorSubcoreMesh(core_axis_name="c", subcore_axis_name="s")
    f = functools.partial(
        pl.kernel,
        mesh=mesh,
        out_type=(
            jax.ShapeDtypeStruct((S_SLOTS, D_MODEL), jnp.float32),
            jax.ShapeDtypeStruct((N_TOKENS,), jnp.int32),
        ),
        compiler_params=pltpu.CompilerParams(needs_layout_passes=False),
        scratch_types=[
            pltpu.VMEM((N_TOKENS,), jnp.int32),
            pltpu.VMEM((NUM_EXPERTS,), jnp.int32),
            pltpu.VMEM((GCHUNK,), jnp.int32),
            pltpu.VMEM((GCHUNK,), jnp.int32),
            pltpu.VMEM((N_TOKENS,), jnp.int32),
            pltpu.VMEM((GSUB, D_MODEL), jnp.float32),
            pltpu.SemaphoreType.DMA,
        ],
    )(_dispatch_body)
    return f(pcode, counts, xf)


def _experts_body(xs_ref, wg_ref, wv_ref, wo_ref, h_ref):
    xe = xs_ref[...]
    wg = wg_ref[0]
    wv = wv_ref[0]
    wo = wo_ref[0]
    g = lax.dot_general(xe, wg, (((1,), (1,)), ((), ())),
                        preferred_element_type=jnp.float32)
    v = lax.dot_general(xe, wv, (((1,), (1,)), ((), ())),
                        preferred_element_type=jnp.float32)
    u = (g / (1.0 + jnp.exp(-g))) * v
    h_ref[...] = lax.dot_general(u, wo, (((1,), (1,)), ((), ())),
                                 preferred_element_type=jnp.float32)


def _experts(xs, w_gate, w_value, w_out):
    return pl.pallas_call(
        _experts_body,
        grid=(NUM_EXPERTS,),
        in_specs=[
            pl.BlockSpec((CAP, D_MODEL), lambda e: (e, 0)),
            pl.BlockSpec((1, EXPERT_DIM, D_MODEL), lambda e: (e, 0, 0)),
            pl.BlockSpec((1, EXPERT_DIM, D_MODEL), lambda e: (e, 0, 0)),
            pl.BlockSpec((1, D_MODEL, EXPERT_DIM), lambda e: (e, 0, 0)),
        ],
        out_specs=pl.BlockSpec((CAP, D_MODEL), lambda e: (e, 0)),
        out_shape=jax.ShapeDtypeStruct((S_SLOTS, D_MODEL), jnp.float32),
        compiler_params=pltpu.CompilerParams(
            dimension_semantics=("arbitrary",)),
    )(xs, w_gate, w_value, w_out)


def _combine_body(slot_hbm, wgt_hbm, h_hbm, out_hbm, pos_v, wgt_v, rows_v, sem):
    wid = lax.axis_index("s") * NC + lax.axis_index("c")
    tb = wid * TOK_PER_W
    pltpu.sync_copy(slot_hbm.at[pl.ds(tb, TOK_PER_W)], pos_v)
    pltpu.sync_copy(wgt_hbm.at[pl.ds(tb, TOK_PER_W)], wgt_v)
    pltpu.async_copy(h_hbm.at[pos_v], rows_v, sem).wait()

    def scale_row(i, carry):
        wv = plsc.load_gather(wgt_v, [jnp.broadcast_to(i, (L,))])
        for j in range(D_MODEL // L):
            sl = pl.ds(j * L, L)
            rows_v[i, sl] = rows_v[i, sl] * wv
        return carry

    lax.fori_loop(0, TOK_PER_W, scale_row, 0)
    pltpu.sync_copy(rows_v, out_hbm.at[pl.ds(tb, TOK_PER_W)])


def _combine(slot, wgt, h):
    mesh = plsc.VectorSubcoreMesh(core_axis_name="c", subcore_axis_name="s")
    f = functools.partial(
        pl.kernel,
        mesh=mesh,
        out_type=jax.ShapeDtypeStruct((N_TOKENS, D_MODEL), jnp.float32),
        compiler_params=pltpu.CompilerParams(needs_layout_passes=False),
        scratch_types=[
            pltpu.VMEM((TOK_PER_W,), jnp.int32),
            pltpu.VMEM((TOK_PER_W,), jnp.float32),
            pltpu.VMEM((TOK_PER_W, D_MODEL), jnp.float32),
            pltpu.SemaphoreType.DMA,
        ],
    )(_combine_body)
    return f(slot, wgt, h)


def kernel(x, gate_w, expert_bias, w_gate, w_value, w_out):
    B_, T_, D_ = x.shape
    xf = x.reshape(T_ * B_, D_)
    pcode, wgt, counts = _router(xf, gate_w, expert_bias)
    xs, slot = _dispatch(pcode, counts, xf)
    h = _experts(xs, w_gate, w_value, w_out)
    out = _combine(slot, wgt, h)
    return out.reshape(B_, T_, D_)
